# Initial kernel scaffold; baseline (speedup 1.0000x reference)
import functools

import jax
import jax.numpy as jnp
from jax import lax
from jax.experimental import pallas as pl
from jax.experimental.pallas import tpu as pltpu
from jax.experimental.pallas import tpu_sc as plsc

_NUM_CORES = 2
_NUM_SUBCORES = 16
_NUM_WORKERS = _NUM_CORES * _NUM_SUBCORES
_LANES = 16

_CHUNK = 100


def _pool(inputs, embed_table):
    B, CTX = inputs.shape
    V, D = embed_table.shape
    assert CTX % _CHUNK == 0
    n_chunks = CTX // _CHUNK
    assert B % _NUM_WORKERS == 0
    b_per_w = B // _NUM_WORKERS
    assert D % _LANES == 0
    d_regs = D // _LANES

    idx3 = inputs.reshape(B, n_chunks, _CHUNK)

    mesh = plsc.VectorSubcoreMesh(
        core_axis_name="c",
        subcore_axis_name="s",
        num_cores=_NUM_CORES,
        num_subcores=_NUM_SUBCORES,
    )

    @functools.partial(
        pl.kernel,
        out_type=jax.ShapeDtypeStruct((B, D), jnp.float32),
        mesh=mesh,
        scratch_types=[
            pltpu.VMEM((2, n_chunks, _CHUNK), jnp.int32),
            pltpu.VMEM((2, CTX, D), jnp.float32),
            pltpu.VMEM((b_per_w, D), jnp.float32),
            pltpu.SemaphoreType.DMA,
            pltpu.SemaphoreType.DMA,
        ],
    )
    def pool_kernel(idx_hbm, table_hbm, out_hbm, idx_v, rows_v, pooled_v,
                    sem0, sem1):
        sems = (sem0, sem1)
        wid = lax.axis_index("s") * _NUM_CORES + lax.axis_index("c")
        base = wid * b_per_w

        def issue(b, d):
            pltpu.sync_copy(idx_hbm.at[base + b], idx_v.at[d])
            for j in range(n_chunks):
                pltpu.async_copy(
                    table_hbm.at[idx_v.at[d, j]],
                    rows_v.at[d, pl.ds(j * _CHUNK, _CHUNK)],
                    sems[d],
                )

        def drain(d):
            for j in range(n_chunks):
                pltpu.make_async_copy(
                    table_hbm.at[idx_v.at[d, j]],
                    rows_v.at[d, pl.ds(j * _CHUNK, _CHUNK)],
                    sems[d],
                ).wait()

        def accumulate(b, d):
            def body(r, carry):
                return tuple(
                    carry[c] + rows_v[d, r, pl.ds(c * _LANES, _LANES)]
                    for c in range(d_regs)
                )
            acc = lax.fori_loop(
                0, CTX, body,
                tuple(jnp.zeros((_LANES,), jnp.float32) for _ in range(d_regs)),
                unroll=2,
            )
            for c in range(d_regs):
                pooled_v[b, pl.ds(c * _LANES, _LANES)] = acc[c]

        for d in range(2):
            issue(d, d)

        def step(i, carry):
            for d in range(2):
                b = 2 * i + d
                drain(d)

                @pl.when(b + 2 < b_per_w)
                def _():
                    issue(b + 2, d)

                accumulate(b, d)
            return carry

        lax.fori_loop(0, b_per_w // 2, step, 0)

        pltpu.sync_copy(pooled_v, out_hbm.at[pl.ds(base, b_per_w)])

    return pool_kernel(idx3, embed_table)


def _linear(pooled, linear_w, linear_b):
    B, D = pooled.shape
    N = linear_w.shape[0]
    N_pad = (N + 127) // 128 * 128
    wt = jnp.pad(linear_w, ((0, N_pad - N), (0, 0))).T
    bias = jnp.pad(linear_b, (0, N_pad - N)).reshape(1, N_pad)

    BM = 1024

    def mm_body(x_ref, w_ref, b_ref, o_ref):
        o_ref[...] = (
            jnp.dot(x_ref[...], w_ref[...], preferred_element_type=jnp.float32)
            + b_ref[...]
        )

    out = pl.pallas_call(
        mm_body,
        grid=(B // BM,),
        in_specs=[
            pl.BlockSpec((BM, D), lambda i: (i, 0)),
            pl.BlockSpec((D, N_pad), lambda i: (0, 0)),
            pl.BlockSpec((1, N_pad), lambda i: (0, 0)),
        ],
        out_specs=pl.BlockSpec((BM, N_pad), lambda i: (i, 0)),
        out_shape=jax.ShapeDtypeStruct((B, N_pad), jnp.float32),
    )(pooled, wt, bias)
    return out[:, :N]


def kernel(inputs, embed_table, linear_w, linear_b):
    inputs = inputs.astype(jnp.int32)
    pooled = _pool(inputs, embed_table)
    return _linear(pooled, linear_w, linear_b)


# speedup vs baseline: 1.8043x; 1.8043x over previous
import functools

import jax
import jax.numpy as jnp
from jax import lax
from jax.experimental import pallas as pl
from jax.experimental.pallas import tpu as pltpu
from jax.experimental.pallas import tpu_sc as plsc

_NUM_CORES = 2
_NUM_SUBCORES = 16
_NUM_WORKERS = _NUM_CORES * _NUM_SUBCORES
_LANES = 16

_CHUNK = 100


def _pool(inputs, embed_table):
    B, CTX = inputs.shape
    V, D = embed_table.shape
    assert CTX % _CHUNK == 0
    n_chunks = CTX // _CHUNK
    assert B % _NUM_WORKERS == 0
    b_per_w = B // _NUM_WORKERS
    assert D % _LANES == 0
    d_regs = D // _LANES

    idx3 = inputs.reshape(B, n_chunks, _CHUNK)

    mesh = plsc.VectorSubcoreMesh(
        core_axis_name="c",
        subcore_axis_name="s",
        num_cores=_NUM_CORES,
        num_subcores=_NUM_SUBCORES,
    )

    @functools.partial(
        pl.kernel,
        out_type=jax.ShapeDtypeStruct((B, D), jnp.float32),
        mesh=mesh,
        scratch_types=[
            pltpu.VMEM((n_chunks, _CHUNK), jnp.int32),
            pltpu.VMEM((CTX, D), jnp.float32),
            pltpu.VMEM((b_per_w, D), jnp.float32),
            pltpu.SemaphoreType.DMA,
        ],
        compiler_params=pltpu.CompilerParams(use_tc_tiling_on_sc=False),
    )
    def pool_kernel(idx_hbm, table_hbm, out_hbm, idx_v, rows_v, pooled_v, sem):
        wid = lax.axis_index("s") * _NUM_CORES + lax.axis_index("c")
        base = wid * b_per_w

        def accumulate(b):
            def body(r, carry):
                return tuple(
                    carry[c] + rows_v[r, pl.ds(c * _LANES, _LANES)]
                    for c in range(d_regs)
                )
            acc = lax.fori_loop(
                0, CTX, body,
                tuple(jnp.zeros((_LANES,), jnp.float32) for _ in range(d_regs)),
                unroll=2,
            )
            for c in range(d_regs):
                pooled_v[b, pl.ds(c * _LANES, _LANES)] = acc[c]

        def step(b, carry):
            pltpu.sync_copy(idx_hbm.at[base + b], idx_v)
            for j in range(n_chunks):
                pltpu.async_copy(
                    table_hbm.at[idx_v.at[j]],
                    rows_v.at[pl.ds(j * _CHUNK, _CHUNK)],
                    sem,
                )
            for j in range(n_chunks):
                pltpu.make_async_copy(
                    table_hbm.at[idx_v.at[j]],
                    rows_v.at[pl.ds(j * _CHUNK, _CHUNK)],
                    sem,
                ).wait()
            accumulate(b)
            return carry

        lax.fori_loop(0, b_per_w, step, 0)

        pltpu.sync_copy(pooled_v, out_hbm.at[pl.ds(base, b_per_w)])

    return pool_kernel(idx3, embed_table)


def _linear(pooled, linear_w, linear_b):
    B, D = pooled.shape
    N = linear_w.shape[0]
    N_pad = (N + 127) // 128 * 128
    wt = jnp.pad(linear_w, ((0, N_pad - N), (0, 0))).T
    bias = jnp.pad(linear_b, (0, N_pad - N)).reshape(1, N_pad)

    BM = 1024

    def mm_body(x_ref, w_ref, b_ref, o_ref):
        o_ref[...] = (
            jnp.dot(x_ref[...], w_ref[...], preferred_element_type=jnp.float32)
            + b_ref[...]
        )

    out = pl.pallas_call(
        mm_body,
        grid=(B // BM,),
        in_specs=[
            pl.BlockSpec((BM, D), lambda i: (i, 0)),
            pl.BlockSpec((D, N_pad), lambda i: (0, 0)),
            pl.BlockSpec((1, N_pad), lambda i: (0, 0)),
        ],
        out_specs=pl.BlockSpec((BM, N_pad), lambda i: (i, 0)),
        out_shape=jax.ShapeDtypeStruct((B, N_pad), jnp.float32),
    )(pooled, wt, bias)
    return out[:, :N]


def kernel(inputs, embed_table, linear_w, linear_b):
    inputs = inputs.astype(jnp.int32)
    pooled = _pool(inputs, embed_table)
    return _linear(pooled, linear_w, linear_b)


# speedup vs baseline: 2.5093x; 1.3908x over previous
import functools

import jax
import jax.numpy as jnp
from jax import lax
from jax.experimental import pallas as pl
from jax.experimental.pallas import tpu as pltpu
from jax.experimental.pallas import tpu_sc as plsc

_NUM_CORES = 2
_NUM_SUBCORES = 16
_NUM_WORKERS = _NUM_CORES * _NUM_SUBCORES
_LANES = 16

_CHUNK = 100


def _pool(inputs, embed_table):
    B, CTX = inputs.shape
    V, D = embed_table.shape
    assert CTX % _CHUNK == 0
    n_chunks = CTX // _CHUNK
    assert B % _NUM_WORKERS == 0
    b_per_w = B // _NUM_WORKERS
    assert D % _LANES == 0
    d_regs = D // _LANES

    idx3 = inputs.reshape(B, n_chunks, _CHUNK)

    mesh = plsc.VectorSubcoreMesh(
        core_axis_name="c",
        subcore_axis_name="s",
        num_cores=_NUM_CORES,
        num_subcores=_NUM_SUBCORES,
    )

    @functools.partial(
        pl.kernel,
        out_type=jax.ShapeDtypeStruct((B, D), jnp.float32),
        mesh=mesh,
        scratch_types=[
            pltpu.VMEM((2, n_chunks, _CHUNK), jnp.int32),
            pltpu.VMEM((2, CTX, D), jnp.float32),
            pltpu.VMEM((b_per_w, D), jnp.float32),
            pltpu.SemaphoreType.DMA,
            pltpu.SemaphoreType.DMA,
        ],
        compiler_params=pltpu.CompilerParams(use_tc_tiling_on_sc=False),
    )
    def pool_kernel(idx_hbm, table_hbm, out_hbm, idx_v, rows_v, pooled_v,
                    sem0, sem1):
        sems = (sem0, sem1)
        wid = lax.axis_index("s") * _NUM_CORES + lax.axis_index("c")
        base = wid * b_per_w

        def issue(b, d):
            pltpu.sync_copy(idx_hbm.at[base + b], idx_v.at[d])
            for j in range(n_chunks):
                pltpu.async_copy(
                    table_hbm.at[idx_v.at[d, j]],
                    rows_v.at[d, pl.ds(j * _CHUNK, _CHUNK)],
                    sems[d],
                )

        def drain(d):
            for j in range(n_chunks):
                pltpu.make_async_copy(
                    table_hbm.at[idx_v.at[d, j]],
                    rows_v.at[d, pl.ds(j * _CHUNK, _CHUNK)],
                    sems[d],
                ).wait()

        def accumulate(b, d):
            def body(r, carry):
                return tuple(
                    carry[c] + rows_v[d, r, pl.ds(c * _LANES, _LANES)]
                    for c in range(d_regs)
                )
            acc = lax.fori_loop(
                0, CTX, body,
                tuple(jnp.zeros((_LANES,), jnp.float32) for _ in range(d_regs)),
                unroll=2,
            )
            for c in range(d_regs):
                pooled_v[b, pl.ds(c * _LANES, _LANES)] = acc[c]

        assert b_per_w % 2 == 0 and b_per_w >= 4
        for d in range(2):
            issue(d, d)

        def step(i, carry):
            for d in range(2):
                b = 2 * i + d
                drain(d)
                accumulate(b, d)
                issue(b + 2, d)
            return carry

        lax.fori_loop(0, b_per_w // 2 - 1, step, 0)

        for d in range(2):
            drain(d)
            accumulate(b_per_w - 2 + d, d)

        pltpu.sync_copy(pooled_v, out_hbm.at[pl.ds(base, b_per_w)])

    return pool_kernel(idx3, embed_table)


def _linear(pooled, linear_w, linear_b):
    B, D = pooled.shape
    N = linear_w.shape[0]
    N_pad = (N + 127) // 128 * 128
    wt = jnp.pad(linear_w, ((0, N_pad - N), (0, 0))).T
    bias = jnp.pad(linear_b, (0, N_pad - N)).reshape(1, N_pad)

    BM = 1024

    def mm_body(x_ref, w_ref, b_ref, o_ref):
        o_ref[...] = (
            jnp.dot(x_ref[...], w_ref[...], preferred_element_type=jnp.float32)
            + b_ref[...]
        )

    out = pl.pallas_call(
        mm_body,
        grid=(B // BM,),
        in_specs=[
            pl.BlockSpec((BM, D), lambda i: (i, 0)),
            pl.BlockSpec((D, N_pad), lambda i: (0, 0)),
            pl.BlockSpec((1, N_pad), lambda i: (0, 0)),
        ],
        out_specs=pl.BlockSpec((BM, N_pad), lambda i: (i, 0)),
        out_shape=jax.ShapeDtypeStruct((B, N_pad), jnp.float32),
    )(pooled, wt, bias)
    return out[:, :N]


def kernel(inputs, embed_table, linear_w, linear_b):
    inputs = inputs.astype(jnp.int32)
    pooled = _pool(inputs, embed_table)
    return _linear(pooled, linear_w, linear_b)


# speedup vs baseline: 3.1865x; 1.2699x over previous
import functools

import jax
import jax.numpy as jnp
from jax import lax
from jax.experimental import pallas as pl
from jax.experimental.pallas import tpu as pltpu
from jax.experimental.pallas import tpu_sc as plsc

_NUM_CORES = 2
_NUM_SUBCORES = 16
_NUM_WORKERS = _NUM_CORES * _NUM_SUBCORES
_LANES = 16

_CHUNK = 100
_BLK = 128
_NBUF = 4


def _pool(inputs, embed_table):
    B, CTX = inputs.shape
    V, D = embed_table.shape
    assert CTX % _CHUNK == 0
    n_chunks = CTX // _CHUNK
    assert B % _NUM_WORKERS == 0
    b_per_w = B // _NUM_WORKERS
    assert D % _LANES == 0
    d_regs = D // _LANES

    idx3 = inputs.reshape(B, n_chunks, _CHUNK)

    mesh = plsc.VectorSubcoreMesh(
        core_axis_name="c",
        subcore_axis_name="s",
        num_cores=_NUM_CORES,
        num_subcores=_NUM_SUBCORES,
    )

    @functools.partial(
        pl.kernel,
        out_type=jax.ShapeDtypeStruct((B, D), jnp.float32),
        mesh=mesh,
        scratch_types=[
            pltpu.VMEM((_BLK, n_chunks, _CHUNK), jnp.int32),
            pltpu.VMEM((_NBUF, CTX, D), jnp.float32),
            pltpu.VMEM((_BLK, D), jnp.float32),
            [pltpu.SemaphoreType.DMA] * _NBUF,
        ],
        compiler_params=pltpu.CompilerParams(use_tc_tiling_on_sc=False),
    )
    def pool_kernel(idx_hbm, table_hbm, out_hbm, idx_v, rows_v, pooled_v,
                    sems):
        wid = lax.axis_index("s") * _NUM_CORES + lax.axis_index("c")
        base = wid * b_per_w
        n_blk = b_per_w // _BLK

        def issue(g, p):
            for j in range(n_chunks):
                pltpu.async_copy(
                    table_hbm.at[idx_v.at[g, j]],
                    rows_v.at[p, pl.ds(j * _CHUNK, _CHUNK)],
                    sems[p],
                )

        def drain(g, p):
            for j in range(n_chunks):
                pltpu.make_async_copy(
                    table_hbm.at[idx_v.at[g, j]],
                    rows_v.at[p, pl.ds(j * _CHUNK, _CHUNK)],
                    sems[p],
                ).wait()

        def accumulate(g, p):
            def body(r, carry):
                return tuple(
                    carry[c] + rows_v[p, r, pl.ds(c * _LANES, _LANES)]
                    for c in range(d_regs)
                )
            acc = lax.fori_loop(
                0, CTX, body,
                tuple(jnp.zeros((_LANES,), jnp.float32) for _ in range(d_regs)),
                unroll=4,
            )
            for c in range(d_regs):
                pooled_v[g, pl.ds(c * _LANES, _LANES)] = acc[c]

        assert _BLK % _NBUF == 0 and _BLK // _NBUF >= 2

        def block(k, carry):
            pltpu.sync_copy(idx_hbm.at[pl.ds(base + k * _BLK, _BLK)], idx_v)
            for p in range(_NBUF):
                issue(p, p)

            def step(i, carry):
                for p in range(_NBUF):
                    g = i * _NBUF + p
                    drain(g, p)
                    accumulate(g, p)
                    issue(g + _NBUF, p)
                return carry

            lax.fori_loop(0, _BLK // _NBUF - 1, step, 0)

            for p in range(_NBUF):
                g = _BLK - _NBUF + p
                drain(g, p)
                accumulate(g, p)

            pltpu.sync_copy(pooled_v,
                            out_hbm.at[pl.ds(base + k * _BLK, _BLK)])
            return carry

        lax.fori_loop(0, n_blk, block, 0)

    return pool_kernel(idx3, embed_table)


def _linear(pooled, linear_w, linear_b):
    B, D = pooled.shape
    N = linear_w.shape[0]
    N_pad = (N + 127) // 128 * 128
    wt = jnp.pad(linear_w, ((0, N_pad - N), (0, 0))).T
    bias = jnp.pad(linear_b, (0, N_pad - N)).reshape(1, N_pad)

    BM = 1024

    def mm_body(x_ref, w_ref, b_ref, o_ref):
        o_ref[...] = (
            jnp.dot(x_ref[...], w_ref[...], preferred_element_type=jnp.float32)
            + b_ref[...]
        )

    out = pl.pallas_call(
        mm_body,
        grid=(B // BM,),
        in_specs=[
            pl.BlockSpec((BM, D), lambda i: (i, 0)),
            pl.BlockSpec((D, N_pad), lambda i: (0, 0)),
            pl.BlockSpec((1, N_pad), lambda i: (0, 0)),
        ],
        out_specs=pl.BlockSpec((BM, N_pad), lambda i: (i, 0)),
        out_shape=jax.ShapeDtypeStruct((B, N_pad), jnp.float32),
    )(pooled, wt, bias)
    return out[:, :N]


def kernel(inputs, embed_table, linear_w, linear_b):
    inputs = inputs.astype(jnp.int32)
    pooled = _pool(inputs, embed_table)
    return _linear(pooled, linear_w, linear_b)


# speedup vs baseline: 3.2494x; 1.0197x over previous
import functools

import jax
import jax.numpy as jnp
from jax import lax
from jax.experimental import pallas as pl
from jax.experimental.pallas import tpu as pltpu
from jax.experimental.pallas import tpu_sc as plsc

_NUM_CORES = 2
_NUM_SUBCORES = 16
_NUM_WORKERS = _NUM_CORES * _NUM_SUBCORES
_LANES = 16

_CHUNKS = (104, 96)
_BLK = 128
_NBUF = 4


def _pool(inputs, embed_table):
    B, CTX = inputs.shape
    V, D = embed_table.shape
    assert sum(_CHUNKS) == CTX
    offs = [sum(_CHUNKS[:j]) for j in range(len(_CHUNKS))]
    assert B % _NUM_WORKERS == 0
    b_per_w = B // _NUM_WORKERS
    assert D % _LANES == 0
    d_regs = D // _LANES


    mesh = plsc.VectorSubcoreMesh(
        core_axis_name="c",
        subcore_axis_name="s",
        num_cores=_NUM_CORES,
        num_subcores=_NUM_SUBCORES,
    )

    @functools.partial(
        pl.kernel,
        out_type=jax.ShapeDtypeStruct((B, D), jnp.float32),
        mesh=mesh,
        scratch_types=[
            pltpu.VMEM((_BLK, CTX), jnp.int32),
            pltpu.VMEM((_NBUF, CTX, D), jnp.float32),
            pltpu.VMEM((_BLK, D), jnp.float32),
            [pltpu.SemaphoreType.DMA] * _NBUF,
        ],
        compiler_params=pltpu.CompilerParams(use_tc_tiling_on_sc=False),
    )
    def pool_kernel(idx_hbm, table_hbm, out_hbm, idx_v, rows_v, pooled_v,
                    sems):
        wid = lax.axis_index("s") * _NUM_CORES + lax.axis_index("c")
        base = wid * b_per_w
        n_blk = b_per_w // _BLK

        def issue(g, p):
            for o, n in zip(offs, _CHUNKS):
                pltpu.async_copy(
                    table_hbm.at[idx_v.at[g, pl.ds(o, n)]],
                    rows_v.at[p, pl.ds(o, n)],
                    sems[p],
                )

        def drain(g, p):
            for o, n in zip(offs, _CHUNKS):
                pltpu.make_async_copy(
                    table_hbm.at[idx_v.at[g, pl.ds(o, n)]],
                    rows_v.at[p, pl.ds(o, n)],
                    sems[p],
                ).wait()

        def accumulate(g, p):
            def body(r, carry):
                return tuple(
                    carry[c] + rows_v[p, r, pl.ds(c * _LANES, _LANES)]
                    for c in range(d_regs)
                )
            acc = lax.fori_loop(
                0, CTX, body,
                tuple(jnp.zeros((_LANES,), jnp.float32) for _ in range(d_regs)),
                unroll=4,
            )
            for c in range(d_regs):
                pooled_v[g, pl.ds(c * _LANES, _LANES)] = acc[c]

        assert _BLK % _NBUF == 0 and _BLK // _NBUF >= 2

        def block(k, carry):
            pltpu.sync_copy(idx_hbm.at[pl.ds(base + k * _BLK, _BLK)], idx_v)
            for p in range(_NBUF):
                issue(p, p)

            def step(i, carry):
                for p in range(_NBUF):
                    g = i * _NBUF + p
                    drain(g, p)
                    accumulate(g, p)
                    issue(g + _NBUF, p)
                return carry

            lax.fori_loop(0, _BLK // _NBUF - 1, step, 0)

            for p in range(_NBUF):
                g = _BLK - _NBUF + p
                drain(g, p)
                accumulate(g, p)

            pltpu.sync_copy(pooled_v,
                            out_hbm.at[pl.ds(base + k * _BLK, _BLK)])
            return carry

        lax.fori_loop(0, n_blk, block, 0)

    return pool_kernel(inputs, embed_table)


def _linear(pooled, linear_w, linear_b):
    B, D = pooled.shape
    N = linear_w.shape[0]
    N_pad = (N + 127) // 128 * 128
    wt = jnp.pad(linear_w, ((0, N_pad - N), (0, 0))).T
    bias = jnp.pad(linear_b, (0, N_pad - N)).reshape(1, N_pad)

    BM = 1024

    def mm_body(x_ref, w_ref, b_ref, o_ref):
        o_ref[...] = (
            jnp.dot(x_ref[...], w_ref[...], preferred_element_type=jnp.float32)
            + b_ref[...]
        )

    out = pl.pallas_call(
        mm_body,
        grid=(B // BM,),
        in_specs=[
            pl.BlockSpec((BM, D), lambda i: (i, 0)),
            pl.BlockSpec((D, N_pad), lambda i: (0, 0)),
            pl.BlockSpec((1, N_pad), lambda i: (0, 0)),
        ],
        out_specs=pl.BlockSpec((BM, N_pad), lambda i: (i, 0)),
        out_shape=jax.ShapeDtypeStruct((B, N_pad), jnp.float32),
    )(pooled, wt, bias)
    return out[:, :N]


def kernel(inputs, embed_table, linear_w, linear_b):
    inputs = inputs.astype(jnp.int32)
    pooled = _pool(inputs, embed_table)
    return _linear(pooled, linear_w, linear_b)
